# baseline (speedup 1.0000x reference)
import functools

import jax
import jax.numpy as jnp
from jax import lax
from jax.experimental import pallas as pl
from jax.experimental.pallas import tpu as pltpu, tpu_sc as plsc

B = 16384
K = 32
R = 1000000
NC = 2
NS = 16
NW = NC * NS
BPW = B // NW
G = 4


def _body(xu_hbm, xi_hbm, wt_hbm, ht_hbm, out_hbm,
          idx_u, idx_i, out_v,
          u0, u1, u2, u3, h0, h1, h2, h3, sem):
  wid = lax.axis_index("s") * NC + lax.axis_index("c")
  base = wid * BPW

  pltpu.sync_copy(xu_hbm.at[pl.ds(base, BPW)], idx_u)
  pltpu.sync_copy(xi_hbm.at[pl.ds(base, BPW)], idx_i)

  ubufs = (u0, u1, u2, u3)
  hbufs = (h0, h1, h2, h3)
  iota = lax.iota(jnp.int32, 16)

  def group(g, _):
    rvu = idx_u[pl.ds(g * 16, 16)]
    rvi = idx_i[pl.ds(g * 16, 16)]
    res = jnp.zeros((16,), jnp.float32)
    for sub in range(4):
      cps = []
      for i in range(G):
        lane = sub * G + i
        cps.append(pltpu.async_copy(
            wt_hbm.at[:, pl.ds((rvu[lane] >> 7) * 128, 128)],
            ubufs[i], sem))
        cps.append(pltpu.async_copy(
            ht_hbm.at[:, pl.ds((rvi[lane] >> 7) * 128, 128)],
            hbufs[i], sem))
      for c in cps:
        c.wait()
      for i in range(G):
        lane = sub * G + i
        cu = jnp.full((16,), rvu[lane] & 127, jnp.int32)
        ci = jnp.full((16,), rvi[lane] & 127, jnp.int32)
        ulo = plsc.load_gather(ubufs[i], [iota, cu])
        uhi = plsc.load_gather(ubufs[i], [iota + 16, cu])
        hlo = plsc.load_gather(hbufs[i], [iota, ci])
        hhi = plsc.load_gather(hbufs[i], [iota + 16, ci])
        s = lax.reduce_sum_p.bind(ulo * hlo + uhi * hhi, axes=(0,))
        res = jnp.where(iota == lane, s, res)
    out_v[pl.ds(g * 16, 16)] = res
    return _

  lax.fori_loop(0, BPW // 16, group, None)

  pltpu.sync_copy(out_v, out_hbm.at[pl.ds(base, BPW)])


def kernel(x_user, x_item, W, H):
  xu = x_user.astype(jnp.int32)
  xi = x_item.astype(jnp.int32)
  wt = jnp.swapaxes(W, 0, 1)
  ht = jnp.swapaxes(H, 0, 1)

  mesh = plsc.VectorSubcoreMesh(core_axis_name="c", subcore_axis_name="s")
  k = functools.partial(
      pl.kernel,
      out_type=jax.ShapeDtypeStruct((B,), jnp.float32),
      mesh=mesh,
      compiler_params=pltpu.CompilerParams(needs_layout_passes=False),
      scratch_types=[
          pltpu.VMEM((BPW,), jnp.int32),
          pltpu.VMEM((BPW,), jnp.int32),
          pltpu.VMEM((BPW,), jnp.float32),
      ] + [pltpu.VMEM((K, 128), jnp.float32)] * 8 + [
          pltpu.SemaphoreType.DMA,
      ],
  )(_body)
  return k(xu, xi, wt, ht)


# speedup vs baseline: 1.2284x; 1.2284x over previous
import functools

import jax
import jax.numpy as jnp
from jax import lax
from jax.experimental import pallas as pl
from jax.experimental.pallas import tpu as pltpu, tpu_sc as plsc

B = 16384
K = 32
R = 1000000
NC = 2
NS = 16
NW = NC * NS
BPW = B // NW

CW = 4096
NCH = 244
BLK = 8 * CW
NCHT = 4 * NCH
CPW = NCHT // NS
TAIL0 = NCH * CW
TAILW = R - TAIL0
TAILP = 640
TAILOFF = NCHT * BLK
FLAT = TAILOFF + K * TAILP


def _copy_body(wt_hbm, ht_hbm, twf_hbm, thf_hbm, wf_hbm, hf_hbm,
               buf0, buf1, tbuf, semr, semw):
  wid = lax.axis_index("s") * NC + lax.axis_index("c")

  bufs = (buf0, buf1)

  for t, src, dst in ((0, wt_hbm, wf_hbm), (1, ht_hbm, hf_hbm)):
    pred = wid < NS if t == 0 else wid >= NS
    w16 = wid - t * NS

    @pl.when(pred)
    def _():
      def run(j, nbuf):
        rds, offs = [], []
        for slot in range(nbuf):
          cid = w16 + NS * (nbuf * j + slot)
          tk = cid // NCH
          c = cid % NCH
          rds.append(pltpu.async_copy(
              src.at[pl.ds(tk * 8, 8), pl.ds(c * CW, CW)], bufs[slot], semr))
          offs.append(cid * BLK)
        ws = []
        for slot in range(nbuf):
          rds[slot].wait()
          for s in range(8):
            ws.append(pltpu.async_copy(
                bufs[slot].at[pl.ds(s, 1), :],
                dst.at[pl.ds(0, 1), pl.ds(offs[slot] + s * CW, CW)], semw))
        for w in ws:
          w.wait()

      def pair(j, _):
        run(j, 2)
        return _

      lax.fori_loop(0, CPW // 2, pair, None)
      run(CPW - 1, 1)

  for tsrc, dst in ((twf_hbm, wf_hbm), (thf_hbm, hf_hbm)):
    pltpu.sync_copy(tsrc.at[pl.ds(0, 1), pl.ds(wid * TAILP, TAILP)], tbuf)
    pltpu.sync_copy(
        tbuf, dst.at[pl.ds(0, 1), pl.ds(wid * TAILP + TAILOFF, TAILP)])


def _gather_body(xu_hbm, xi_hbm, wf_hbm, hf_hbm, out_hbm,
                 idx_u, idx_i, adr_u, adr_i, u_kbuf, h_kbuf, out_v, sem):
  wid = lax.axis_index("s") * NC + lax.axis_index("c")
  base = wid * BPW

  pltpu.sync_copy(xu_hbm.at[pl.ds(base, BPW)], idx_u)
  pltpu.sync_copy(xi_hbm.at[pl.ds(base, BPW)], idx_i)

  kc_main = [(k // 8) * NCH * BLK + (k % 8) * CW for k in range(K)]
  kc_tail = [TAILOFF + k * TAILP - TAIL0 for k in range(K)]

  def addr_group(g, _):
    col = pl.ds(g * 16, 16)
    for src, dst in ((idx_u, adr_u), (idx_i, adr_i)):
      r = src[col]
      bmain = ((r >> 12) << 15) + (r & (CW - 1))
      in_main = r < TAIL0
      for k in range(K):
        dst[k, col] = jnp.where(in_main, bmain + kc_main[k], r + kc_tail[k])
    return _

  lax.fori_loop(0, BPW // 16, addr_group, None)

  copies = []
  for k in range(K):
    copies.append(pltpu.async_copy(
        wf_hbm.at[0].at[adr_u.at[k]], u_kbuf.at[k], sem))
    copies.append(pltpu.async_copy(
        hf_hbm.at[0].at[adr_i.at[k]], h_kbuf.at[k], sem))
  for c in copies:
    c.wait()

  iota = lax.iota(jnp.int32, 16)

  def group(g, _):
    col = pl.ds(g * 16, 16)
    acc = jnp.zeros((16,), jnp.float32)
    for k in range(K):
      acc = acc + u_kbuf[k, col] * h_kbuf[k, col]
    plsc.store_scatter(out_v, [g * 16 + iota], acc)
    return _

  lax.fori_loop(0, BPW // 16, group, None)

  pltpu.sync_copy(out_v, out_hbm.at[pl.ds(base, BPW)])


def kernel(x_user, x_item, W, H):
  xu = x_user.astype(jnp.int32)
  xi = x_item.astype(jnp.int32)
  wt = jnp.swapaxes(W, 0, 1)
  ht = jnp.swapaxes(H, 0, 1)
  pad = ((0, 0), (0, TAILP - TAILW))
  twf = jnp.pad(wt[:, TAIL0:], pad).reshape(1, -1)
  thf = jnp.pad(ht[:, TAIL0:], pad).reshape(1, -1)

  mesh = plsc.VectorSubcoreMesh(core_axis_name="c", subcore_axis_name="s")

  copyk = functools.partial(
      pl.kernel,
      out_type=(jax.ShapeDtypeStruct((1, FLAT), jnp.float32),
                jax.ShapeDtypeStruct((1, FLAT), jnp.float32)),
      mesh=mesh,
      compiler_params=pltpu.CompilerParams(needs_layout_passes=False),
      scratch_types=[
          pltpu.VMEM((8, CW), jnp.float32),
          pltpu.VMEM((8, CW), jnp.float32),
          pltpu.VMEM((1, TAILP), jnp.float32),
          pltpu.SemaphoreType.DMA,
          pltpu.SemaphoreType.DMA,
      ],
  )(_copy_body)
  wf, hf = copyk(wt, ht, twf, thf)

  gather = functools.partial(
      pl.kernel,
      out_type=jax.ShapeDtypeStruct((B,), jnp.float32),
      mesh=mesh,
      compiler_params=pltpu.CompilerParams(
          needs_layout_passes=False, use_tc_tiling_on_sc=False),
      scratch_types=[
          pltpu.VMEM((BPW,), jnp.int32),
          pltpu.VMEM((BPW,), jnp.int32),
          pltpu.VMEM((K, BPW), jnp.int32),
          pltpu.VMEM((K, BPW), jnp.int32),
          pltpu.VMEM((K, BPW), jnp.float32),
          pltpu.VMEM((K, BPW), jnp.float32),
          pltpu.VMEM((BPW,), jnp.float32),
          pltpu.SemaphoreType.DMA,
      ],
  )(_gather_body)
  return gather(xu, xi, wf, hf)
